# baseline (speedup 1.0000x reference)
import functools

import jax
import jax.numpy as jnp
from jax import lax
from jax.experimental import pallas as pl
from jax.experimental.pallas import tpu as pltpu
from jax.experimental.pallas import tpu_sc as plsc

SEQ = 2048
D_MODEL = 768
D_FF = 1024
NUM_EXPERTS = 64
BT = 128
L = 16
NW = 32
TPW = SEQ // NW
VPW = TPW // L
NVREG = SEQ // L

_SC_MESH = plsc.VectorSubcoreMesh(core_axis_name="c", subcore_axis_name="s")


def _dispatch_body(idx_hbm, x_hbm, w_hbm,
                   xs_hbm, ws_hbm, dest_hbm, counts_hbm, offs_hbm,
                   idx_all, ev, hist, offs_v, rank_own, dest_v, rows_v, wv,
                   sem):
    wid = lax.axis_index("s") * 2 + lax.axis_index("c")
    lane = lax.iota(jnp.int32, 16)

    pltpu.sync_copy(idx_hbm, idx_all)
    for kk in range(NUM_EXPERTS // L):
        hist[pl.ds(kk * L, L)] = jnp.zeros((L,), jnp.int32)

    own0 = wid * VPW

    def body(k, _):
        e_k = idx_all[pl.ds(k * L, L)]
        ev[pl.ds(L, L)] = e_k
        rank = jnp.zeros((L,), jnp.int32)
        tot = jnp.zeros((L,), jnp.int32)
        for j in range(L):
            bj = plsc.load_gather(ev, [jnp.full((L,), L + j, jnp.int32)])
            m = e_k == bj
            rank = rank + jnp.where(m & (lane > j), 1, 0)
            tot = tot + jnp.where(m, 1, 0)
        base_e = plsc.load_gather(hist, [e_k])

        @pl.when((k >= own0) & (k < own0 + VPW))
        def _():
            rank_own[pl.ds((k - own0) * L, L)] = base_e + rank

        plsc.store_scatter(hist, [e_k], base_e + tot)
        return 0

    lax.fori_loop(0, NVREG, body, 0)

    carry = jnp.int32(0)
    for kk in range(NUM_EXPERTS // L):
        c = hist[pl.ds(kk * L, L)]
        offs_v[pl.ds(kk * L, L)] = plsc.cumsum(c) - c + carry
        carry = carry + jnp.sum(c)

    for kk in range(VPW):
        e_k = idx_all[pl.ds((own0 + kk) * L, L)]
        off_e = plsc.load_gather(offs_v, [e_k])
        dest_v[pl.ds(kk * L, L)] = off_e + rank_own[pl.ds(kk * L, L)]

    base = wid * TPW
    pltpu.sync_copy(dest_v, dest_hbm.at[pl.ds(base, TPW)])
    pltpu.sync_copy(x_hbm.at[pl.ds(base, TPW)], rows_v)
    pltpu.async_copy(rows_v, xs_hbm.at[dest_v], sem).wait()
    pltpu.sync_copy(w_hbm.at[pl.ds(base, TPW)], wv)
    pltpu.async_copy(wv, ws_hbm.at[dest_v], sem).wait()

    @pl.when(wid == 0)
    def _():
        pltpu.sync_copy(hist, counts_hbm)
        pltpu.sync_copy(offs_v, offs_hbm)


@functools.partial(
    pl.kernel,
    mesh=_SC_MESH,
    out_type=[
        jax.ShapeDtypeStruct((SEQ, D_MODEL), jnp.float32),
        jax.ShapeDtypeStruct((SEQ,), jnp.float32),
        jax.ShapeDtypeStruct((SEQ,), jnp.int32),
        jax.ShapeDtypeStruct((NUM_EXPERTS,), jnp.int32),
        jax.ShapeDtypeStruct((NUM_EXPERTS,), jnp.int32),
    ],
    scratch_types=[
        pltpu.VMEM((SEQ,), jnp.int32),
        pltpu.VMEM((2 * L,), jnp.int32),
        pltpu.VMEM((NUM_EXPERTS,), jnp.int32),
        pltpu.VMEM((NUM_EXPERTS,), jnp.int32),
        pltpu.VMEM((TPW,), jnp.int32),
        pltpu.VMEM((TPW,), jnp.int32),
        pltpu.VMEM((TPW, D_MODEL), jnp.float32),
        pltpu.VMEM((TPW,), jnp.float32),
        pltpu.SemaphoreType.DMA,
    ],
    compiler_params=pltpu.CompilerParams(needs_layout_passes=False),
)
def _sc_dispatch(idx_hbm, x_hbm, w_hbm, *rest):
    _dispatch_body(idx_hbm, x_hbm, w_hbm, *rest)


@functools.partial(
    pl.kernel,
    mesh=_SC_MESH,
    out_type=jax.ShapeDtypeStruct((SEQ, D_MODEL), jnp.float32),
    scratch_types=[
        pltpu.VMEM((TPW,), jnp.int32),
        pltpu.VMEM((TPW, D_MODEL), jnp.float32),
        pltpu.SemaphoreType.DMA,
    ],
    compiler_params=pltpu.CompilerParams(needs_layout_passes=False),
)
def _sc_combine(ys_hbm, dest_hbm, out_hbm, dest_v, rows_v, sem):
    wid = lax.axis_index("s") * 2 + lax.axis_index("c")
    base = wid * TPW
    pltpu.sync_copy(dest_hbm.at[pl.ds(base, TPW)], dest_v)
    pltpu.async_copy(ys_hbm.at[dest_v], rows_v, sem).wait()
    pltpu.sync_copy(rows_v, out_hbm.at[pl.ds(base, TPW)])


def _mlp_kernel(offs_ref, cnts_ref, xs_ref, ws_ref, w1_ref, w2_ref, ys_ref):
    e = pl.program_id(0)
    start = offs_ref[e]
    cnt = cnts_ref[e]
    t0 = start // BT
    t1 = lax.div(start + cnt + BT - 1, BT)
    w1e = w1_ref[0]
    w2e = w2_ref[0]

    def body(j, _):
        s = (t0 + j) * BT
        xb = xs_ref[pl.ds(s, BT), :]
        h = jnp.dot(xb, w1e, preferred_element_type=jnp.float32)
        h = h * jax.nn.sigmoid(h)
        yb = jnp.dot(h, w2e, preferred_element_type=jnp.float32)
        yb = yb * ws_ref[pl.ds(s, BT), :]
        rid = s + lax.broadcasted_iota(jnp.int32, (BT, 1), 0)
        mask = (rid >= start) & (rid < start + cnt)
        ys_ref[pl.ds(s, BT), :] = jnp.where(mask, yb, ys_ref[pl.ds(s, BT), :])
        return 0

    lax.fori_loop(0, t1 - t0, body, 0)


def _grouped_mlp(xs, ws, offsets, counts, w1, w2):
    grid_spec = pltpu.PrefetchScalarGridSpec(
        num_scalar_prefetch=2,
        grid=(NUM_EXPERTS,),
        in_specs=[
            pl.BlockSpec((SEQ, D_MODEL), lambda e, o, c: (0, 0)),
            pl.BlockSpec((SEQ, 1), lambda e, o, c: (0, 0)),
            pl.BlockSpec((1, D_MODEL, D_FF), lambda e, o, c: (e, 0, 0)),
            pl.BlockSpec((1, D_FF, D_MODEL), lambda e, o, c: (e, 0, 0)),
        ],
        out_specs=pl.BlockSpec((SEQ, D_MODEL), lambda e, o, c: (0, 0)),
    )
    return pl.pallas_call(
        _mlp_kernel,
        grid_spec=grid_spec,
        out_shape=jax.ShapeDtypeStruct((SEQ, D_MODEL), jnp.float32),
        compiler_params=pltpu.CompilerParams(
            dimension_semantics=("arbitrary",),
        ),
    )(offsets, counts, xs, ws, w1, w2)


def kernel(x, expert_weights, expert_indices, w1, w2):
    flat_x = x.reshape(SEQ, D_MODEL)
    idx = expert_indices.reshape(SEQ).astype(jnp.int32)
    wflat = expert_weights.reshape(SEQ)

    xs, ws, dest, counts, offsets = _sc_dispatch(idx, flat_x, wflat)
    ys = _grouped_mlp(xs, ws[:, None], offsets, counts, w1, w2)
    out_flat = _sc_combine(ys, dest)
    return out_flat.reshape(x.shape), counts
